# Initial kernel scaffold; baseline (speedup 1.0000x reference)
import jax
import jax.numpy as jnp
from jax.experimental import pallas as pl


def kernel(x, edge_index, Wl1, bl1, Wr1, Wl, bl, Wr, bn_g, bn_b, lin_W, lin_b, bn6_g, bn6_b, out_W, out_b):
    raise NotImplementedError("write your pallas kernel here")


# speedup vs baseline: 7.7113x; 7.7113x over previous
import functools

import jax
import jax.numpy as jnp
import numpy as np
from jax import lax
from jax.experimental import pallas as pl
from jax.experimental.pallas import tpu as pltpu
from jax.experimental.pallas import tpu_sc as plsc

F = 16
NC = 1
NS = 16
NW = NC * NS
EB = 128
KB = 56


@functools.lru_cache(maxsize=None)
def _make_agg(n_pad, rows_per_w, stripe):
    mesh = plsc.VectorSubcoreMesh(core_axis_name="c", subcore_axis_name="s",
                                  num_cores=NC)
    nblk = rows_per_w // KB

    @functools.partial(
        pl.kernel,
        out_type=jax.ShapeDtypeStruct((n_pad, F), jnp.float32),
        mesh=mesh,
        scratch_types=[
            pltpu.VMEM((KB, EB), jnp.int32),
            pltpu.VMEM((KB, EB), jnp.int32),
            pltpu.VMEM((EB, F), jnp.float32),
            pltpu.VMEM_SHARED((n_pad, F), jnp.float32),
            pltpu.SemaphoreType.DMA,
        ],
        compiler_params=pltpu.CompilerParams(use_tc_tiling_on_sc=False),
    )
    def agg(h_hbm, src_hbm, dst_hbm, zero_hbm, out_hbm,
            src_v, dst_v, rows_v, acc_sh, sem):
        s = lax.axis_index("s")
        pltpu.sync_copy(zero_hbm, acc_sh.at[pl.ds(s * stripe, stripe)])
        plsc.subcore_barrier()
        row0 = s * rows_per_w

        def blk_body(b, carry):
            pltpu.sync_copy(src_hbm.at[pl.ds(row0 + b * KB, KB)], src_v)
            pltpu.sync_copy(dst_hbm.at[pl.ds(row0 + b * KB, KB)], dst_v)

            def body(j, c2):
                pltpu.async_copy(h_hbm.at[src_v.at[j]], rows_v, sem).wait()
                pltpu.sync_copy(rows_v, acc_sh.at[dst_v.at[j]], add=True)
                return c2

            lax.fori_loop(0, KB, body, 0)
            return carry

        lax.fori_loop(0, nblk, blk_body, 0)
        plsc.subcore_barrier()
        pltpu.sync_copy(acc_sh.at[pl.ds(s * stripe, stripe)],
                        out_hbm.at[pl.ds(s * stripe, stripe)])

    return agg


def _aggregate(h16, src_rows, dst_rows, zeros_stripe, n_pad):
    rows_total = src_rows.shape[0]
    agg = _make_agg(n_pad, rows_total // NW, n_pad // NS)
    return agg(h16, src_rows, dst_rows, zeros_stripe)


def _conv_dense(agg2, h, Wl16, bl16, Wr16, stats, g16, b16, first, need_stats):
    n = h.shape[0]
    R = 10000
    G = n // R
    f32 = jnp.float32

    def body(*refs):
        if first:
            agg_ref, h_ref, Wl_ref, bl_ref, Wr_ref = refs[:5]
            orefs = refs[5:]
        else:
            agg_ref, h_ref, Wl_ref, bl_ref, Wr_ref, st_ref, g_ref, b_ref = refs[:8]
            orefs = refs[8:]
        if need_stats:
            out_ref, stats_ref, acc_ref = orefs
        else:
            (out_ref,) = orefs
        i = pl.program_id(0)
        agg = agg_ref[...]
        deg = agg[:, 12:13]
        mean = agg * (1.0 / jnp.maximum(deg, 1.0))
        if first:
            hn = h_ref[...]
            meanc = mean
        else:
            st = st_ref[...]
            m = st[0:1, :] * (1.0 / n)
            v = st[1:2, :] * (1.0 / n) - m * m
            sc = g_ref[...] * lax.rsqrt(v + 1e-5)
            tc = b_ref[...] - m * sc
            hn = h_ref[...] * sc + tc
            meanc = mean * sc + tc * jnp.minimum(deg, 1.0)
        z = (jnp.dot(meanc, Wl_ref[...], preferred_element_type=f32)
             + bl_ref[...]
             + jnp.dot(hn, Wr_ref[...], preferred_element_type=f32))
        hp = jnp.maximum(z, 0.0)
        out_ref[...] = hp
        if need_stats:
            @pl.when(i == 0)
            def _():
                acc_ref[...] = jnp.zeros_like(acc_ref)
            acc_ref[0:1, :] += jnp.sum(hp, axis=0, keepdims=True)
            acc_ref[1:2, :] += jnp.sum(hp * hp, axis=0, keepdims=True)
            @pl.when(i == G - 1)
            def _():
                stats_ref[...] = acc_ref[...]

    in_specs = [
        pl.BlockSpec((R, F), lambda i: (i, 0)),
        pl.BlockSpec((R, F), lambda i: (i, 0)),
        pl.BlockSpec((F, F), lambda i: (0, 0)),
        pl.BlockSpec((1, F), lambda i: (0, 0)),
        pl.BlockSpec((F, F), lambda i: (0, 0)),
    ]
    args = [agg2, h, Wl16, bl16, Wr16]
    if not first:
        in_specs += [pl.BlockSpec((8, F), lambda i: (0, 0)),
                     pl.BlockSpec((1, F), lambda i: (0, 0)),
                     pl.BlockSpec((1, F), lambda i: (0, 0))]
        args += [stats, g16, b16]
    out_shape = [jax.ShapeDtypeStruct((n, F), f32)]
    out_specs = [pl.BlockSpec((R, F), lambda i: (i, 0))]
    scratch = []
    if need_stats:
        out_shape.append(jax.ShapeDtypeStruct((8, F), f32))
        out_specs.append(pl.BlockSpec((8, F), lambda i: (0, 0)))
        scratch.append(pltpu.VMEM((8, F), f32))
    res = pl.pallas_call(
        body,
        grid=(G,),
        in_specs=in_specs,
        out_specs=out_specs if len(out_specs) > 1 else out_specs[0],
        out_shape=out_shape if len(out_shape) > 1 else out_shape[0],
        scratch_shapes=scratch,
    )(*args)
    return res


def _head(hin, W16, bh16, P16, stats, g16, b16, Wo16, bo2, mode):
    n = hin.shape[0]
    R = 10000
    G = n // R
    f32 = jnp.float32
    two_n = 2.0 * n

    def body(*refs):
        if mode == 0:
            h_ref, W_ref, bh_ref = refs[:3]
            orefs = refs[3:]
        elif mode == 1:
            h_ref, W_ref, bh_ref, P_ref, st_ref, g_ref, b_ref = refs[:7]
            orefs = refs[7:]
        else:
            h_ref, P_ref, st_ref, g_ref, b_ref, Wo_ref, bo_ref = refs[:7]
            orefs = refs[7:]
        i = pl.program_id(0)
        if mode == 0:
            r = h_ref[...]
        else:
            st = st_ref[...]
            sumf = jnp.dot(st[0:1, :], P_ref[...], preferred_element_type=f32)
            sqf = jnp.dot(st[1:2, :], P_ref[...], preferred_element_type=f32)
            m = sumf * (1.0 / two_n)
            v = sqf * (1.0 / two_n) - m * m
            S = g_ref[...] * lax.rsqrt(v + 1e-5)
            T = b_ref[...] - m * S
            r = jnp.maximum(h_ref[...] * S + T, 0.0)
        if mode == 2:
            out_ref = orefs[0]
            o = jnp.dot(r, Wo_ref[...], preferred_element_type=f32) + bo_ref[...]
            out_ref[...] = jax.nn.sigmoid(o)
        else:
            out_ref, stats_ref, acc_ref = orefs
            u = jnp.dot(r, W_ref[...], preferred_element_type=f32) + bh_ref[...]
            out_ref[...] = u
            @pl.when(i == 0)
            def _():
                acc_ref[...] = jnp.zeros_like(acc_ref)
            acc_ref[0:1, :] += jnp.sum(u, axis=0, keepdims=True)
            acc_ref[1:2, :] += jnp.sum(u * u, axis=0, keepdims=True)
            @pl.when(i == G - 1)
            def _():
                stats_ref[...] = acc_ref[...]

    full = lambda shape: pl.BlockSpec(shape, lambda i: tuple(0 for _ in shape))
    hspec = pl.BlockSpec((R, F), lambda i: (i, 0))
    if mode == 0:
        in_specs = [hspec, full((F, F)), full((1, F))]
        args = [hin, W16, bh16]
    elif mode == 1:
        in_specs = [hspec, full((F, F)), full((1, F)), full((F, F)),
                    full((8, F)), full((1, F)), full((1, F))]
        args = [hin, W16, bh16, P16, stats, g16, b16]
    else:
        in_specs = [hspec, full((F, F)), full((8, F)), full((1, F)),
                    full((1, F)), full((F, 8)), full((1, 8))]
        args = [hin, P16, stats, g16, b16, Wo16, bo2]
    if mode == 2:
        out_shape = jax.ShapeDtypeStruct((n, 8), f32)
        out_specs = pl.BlockSpec((R, 8), lambda i: (i, 0))
        scratch = []
    else:
        out_shape = [jax.ShapeDtypeStruct((n, F), f32),
                     jax.ShapeDtypeStruct((8, F), f32)]
        out_specs = [pl.BlockSpec((R, F), lambda i: (i, 0)),
                     pl.BlockSpec((8, F), lambda i: (0, 0))]
        scratch = [pltpu.VMEM((8, F), f32)]
    return pl.pallas_call(
        body,
        grid=(G,),
        in_specs=in_specs,
        out_specs=out_specs,
        out_shape=out_shape,
        scratch_shapes=scratch,
    )(*args)


def kernel(x, edge_index, Wl1, bl1, Wr1, Wl, bl, Wr, bn_g, bn_b,
           lin_W, lin_b, bn6_g, bn6_b, out_W, out_b):
    f32 = jnp.float32
    n = x.shape[0]
    e = edge_index.shape[1]
    n_pad = (n // 128 + 2) * 128
    rows_per_w = ((-(-e // (NW * EB)) + KB - 1) // KB) * KB
    rows_total = rows_per_w * NW
    e_pad = rows_total * EB

    src = edge_index[0]
    dst = edge_index[1]
    src_rows = jnp.concatenate(
        [src, jnp.zeros((e_pad - e,), jnp.int32)]).reshape(rows_total, EB)
    dst_rows = jnp.concatenate(
        [dst, jnp.full((e_pad - e,), n, jnp.int32)]).reshape(rows_total, EB)
    zeros_stripe = jnp.zeros((n_pad // NS, F), f32)

    ecol = jnp.zeros((n, 1), f32)
    h16 = jnp.concatenate(
        [x, jnp.zeros((n, 12 - x.shape[1]), f32), 1.0 + ecol,
         jnp.zeros((n, 3), f32)], axis=1)

    def pad_w(w):
        return jnp.pad(w, ((0, F - w.shape[0]), (0, F - w.shape[1])))

    def pad_b(b, col12=0.0):
        v = jnp.pad(b, (0, F - b.shape[0]))
        return v.at[12].set(col12).reshape(1, F)

    def pad_v(v):
        return jnp.pad(v, (0, F - v.shape[0])).reshape(1, F)

    stats = None
    for i in range(8):
        agg2 = _aggregate(h16, src_rows, dst_rows, zeros_stripe, n_pad)
        if i == 0:
            Wl16, bl16, Wr16 = pad_w(Wl1), pad_b(bl1, 1.0), pad_w(Wr1)
            g16 = b16 = None
        else:
            Wl16 = pad_w(Wl[i - 1])
            bl16 = pad_b(bl[i - 1], 1.0)
            Wr16 = pad_w(Wr[i - 1])
            g16 = pad_v(bn_g[i - 1])
            b16 = pad_v(bn_b[i - 1])
        res = _conv_dense(agg2, h16, Wl16, bl16, Wr16, stats, g16, b16,
                          first=(i == 0), need_stats=(i < 7))
        if i < 7:
            h16, stats = res
        else:
            h16 = res

    Wh = np.zeros((F, F), np.float32)
    Wh16 = jnp.asarray(Wh).at[0:6, 0:6].set(lin_W).at[6:12, 6:12].set(lin_W)
    bh16 = jnp.concatenate([lin_b, lin_b, jnp.zeros((4,), f32)]).reshape(1, F)
    P = np.zeros((F, F), np.float32)
    for a in range(12):
        for bb in range(12):
            if a % 6 == bb % 6:
                P[a, bb] = 1.0
    P16 = jnp.asarray(P)
    g6_16 = jnp.concatenate([bn6_g, bn6_g, jnp.zeros((4,), f32)]).reshape(1, F)
    b6_16 = jnp.concatenate([bn6_b, bn6_b, jnp.zeros((4,), f32)]).reshape(1, F)
    Wo16 = jnp.zeros((F, 8), f32).at[0:6, 0:1].set(out_W).at[6:12, 1:2].set(out_W)
    bo2 = jnp.zeros((1, 8), f32).at[0, 0].set(out_b[0]).at[0, 1].set(out_b[0])

    u, hstats = _head(h16, Wh16, bh16, P16, None, None, None, None, None, mode=0)
    u, hstats = _head(u, Wh16, bh16, P16, hstats, g6_16, b6_16, None, None, mode=1)
    u, hstats = _head(u, Wh16, bh16, P16, hstats, g6_16, b6_16, None, None, mode=1)
    o = _head(u, None, None, P16, hstats, g6_16, b6_16, Wo16, bo2, mode=2)
    return o[:, 0:2].reshape(2 * n, 1)


# speedup vs baseline: 17.1072x; 2.2185x over previous
import functools

import jax
import jax.numpy as jnp
import numpy as np
from jax import lax
from jax.experimental import pallas as pl
from jax.experimental.pallas import tpu as pltpu
from jax.experimental.pallas import tpu_sc as plsc

F = 16
NC = 1
NS = 16
NW = NC * NS
EB = 128
KB = 56


CH = 512
U = 3


@functools.lru_cache(maxsize=None)
def _make_agg(n_pad, ew, stripe):
    mesh = plsc.VectorSubcoreMesh(core_axis_name="c", subcore_axis_name="s",
                                  num_cores=NC)
    nit = ew // CH
    nbody = nit // U
    ntail = nit - nbody * U

    @functools.partial(
        pl.kernel,
        out_type=jax.ShapeDtypeStruct((n_pad, F), jnp.float32),
        mesh=mesh,
        scratch_types=[
            pltpu.VMEM((U, CH), jnp.int32),
            pltpu.VMEM((U, CH), jnp.int32),
            pltpu.VMEM((U, CH, F), jnp.float32),
            pltpu.VMEM_SHARED((n_pad, F), jnp.float32),
        ] + [pltpu.SemaphoreType.DMA] * (3 * U),
        compiler_params=pltpu.CompilerParams(use_tc_tiling_on_sc=False),
    )
    def agg(h_hbm, src_hbm, dst_hbm, zero_hbm, out_hbm,
            src_v, dst_v, rows_v, acc_sh, *sems):
        semi = sems[0:U]
        semg = sems[U:2 * U]
        sems_ = sems[2 * U:3 * U]
        s = lax.axis_index("s")
        pltpu.sync_copy(zero_hbm, acc_sh.at[pl.ds(s * stripe, stripe)])
        plsc.subcore_barrier()
        base = s * ew

        def chunks(off, m):
            di = []
            for u in range(m):
                o = off + u * CH
                di.append((
                    pltpu.async_copy(src_hbm.at[pl.ds(o, CH)], src_v.at[u],
                                     semi[u]),
                    pltpu.async_copy(dst_hbm.at[pl.ds(o, CH)], dst_v.at[u],
                                     semi[u])))
            gd = []
            for u in range(m):
                di[u][0].wait()
                gd.append(pltpu.async_copy(h_hbm.at[src_v.at[u]],
                                           rows_v.at[u], semg[u]))
            sd = []
            for u in range(m):
                gd[u].wait()
                di[u][1].wait()
                sd.append(pltpu.async_copy(rows_v.at[u],
                                           acc_sh.at[dst_v.at[u]],
                                           sems_[u], add=True))
            for u in range(m):
                sd[u].wait()

        def body(ii, carry):
            chunks(base + ii * (U * CH), U)
            return carry

        lax.fori_loop(0, nbody, body, 0)
        if ntail:
            chunks(base + nbody * (U * CH), ntail)
        plsc.subcore_barrier()
        pltpu.sync_copy(acc_sh.at[pl.ds(s * stripe, stripe)],
                        out_hbm.at[pl.ds(s * stripe, stripe)])

    return agg


def _aggregate(h16, src_flat, dst_flat, zeros_stripe, n_pad):
    e_pad = src_flat.shape[0]
    agg = _make_agg(n_pad, e_pad // NW, n_pad // NS)
    return agg(h16, src_flat, dst_flat, zeros_stripe)


def _conv_dense(agg2, h, Wl16, bl16, Wr16, stats, g16, b16, first, need_stats):
    n = h.shape[0]
    R = 10000
    G = n // R
    f32 = jnp.float32

    def body(*refs):
        if first:
            agg_ref, h_ref, Wl_ref, bl_ref, Wr_ref = refs[:5]
            orefs = refs[5:]
        else:
            agg_ref, h_ref, Wl_ref, bl_ref, Wr_ref, st_ref, g_ref, b_ref = refs[:8]
            orefs = refs[8:]
        if need_stats:
            out_ref, stats_ref, acc_ref = orefs
        else:
            (out_ref,) = orefs
        i = pl.program_id(0)
        agg = agg_ref[...]
        deg = agg[:, 12:13]
        mean = agg * (1.0 / jnp.maximum(deg, 1.0))
        if first:
            hn = h_ref[...]
            meanc = mean
        else:
            st = st_ref[...]
            m = st[0:1, :] * (1.0 / n)
            v = st[1:2, :] * (1.0 / n) - m * m
            sc = g_ref[...] * lax.rsqrt(v + 1e-5)
            tc = b_ref[...] - m * sc
            hn = h_ref[...] * sc + tc
            meanc = mean * sc + tc * jnp.minimum(deg, 1.0)
        z = (jnp.dot(meanc, Wl_ref[...], preferred_element_type=f32)
             + bl_ref[...]
             + jnp.dot(hn, Wr_ref[...], preferred_element_type=f32))
        hp = jnp.maximum(z, 0.0)
        out_ref[...] = hp
        if need_stats:
            @pl.when(i == 0)
            def _():
                acc_ref[...] = jnp.zeros_like(acc_ref)
            acc_ref[0:1, :] += jnp.sum(hp, axis=0, keepdims=True)
            acc_ref[1:2, :] += jnp.sum(hp * hp, axis=0, keepdims=True)
            @pl.when(i == G - 1)
            def _():
                stats_ref[...] = acc_ref[...]

    in_specs = [
        pl.BlockSpec((R, F), lambda i: (i, 0)),
        pl.BlockSpec((R, F), lambda i: (i, 0)),
        pl.BlockSpec((F, F), lambda i: (0, 0)),
        pl.BlockSpec((1, F), lambda i: (0, 0)),
        pl.BlockSpec((F, F), lambda i: (0, 0)),
    ]
    args = [agg2, h, Wl16, bl16, Wr16]
    if not first:
        in_specs += [pl.BlockSpec((8, F), lambda i: (0, 0)),
                     pl.BlockSpec((1, F), lambda i: (0, 0)),
                     pl.BlockSpec((1, F), lambda i: (0, 0))]
        args += [stats, g16, b16]
    out_shape = [jax.ShapeDtypeStruct((n, F), f32)]
    out_specs = [pl.BlockSpec((R, F), lambda i: (i, 0))]
    scratch = []
    if need_stats:
        out_shape.append(jax.ShapeDtypeStruct((8, F), f32))
        out_specs.append(pl.BlockSpec((8, F), lambda i: (0, 0)))
        scratch.append(pltpu.VMEM((8, F), f32))
    res = pl.pallas_call(
        body,
        grid=(G,),
        in_specs=in_specs,
        out_specs=out_specs if len(out_specs) > 1 else out_specs[0],
        out_shape=out_shape if len(out_shape) > 1 else out_shape[0],
        scratch_shapes=scratch,
    )(*args)
    return res


def _head(hin, W16, bh16, P16, stats, g16, b16, Wo16, bo2, mode):
    n = hin.shape[0]
    R = 10000
    G = n // R
    f32 = jnp.float32
    two_n = 2.0 * n

    def body(*refs):
        if mode == 0:
            h_ref, W_ref, bh_ref = refs[:3]
            orefs = refs[3:]
        elif mode == 1:
            h_ref, W_ref, bh_ref, P_ref, st_ref, g_ref, b_ref = refs[:7]
            orefs = refs[7:]
        else:
            h_ref, P_ref, st_ref, g_ref, b_ref, Wo_ref, bo_ref = refs[:7]
            orefs = refs[7:]
        i = pl.program_id(0)
        if mode == 0:
            r = h_ref[...]
        else:
            st = st_ref[...]
            sumf = jnp.dot(st[0:1, :], P_ref[...], preferred_element_type=f32)
            sqf = jnp.dot(st[1:2, :], P_ref[...], preferred_element_type=f32)
            m = sumf * (1.0 / two_n)
            v = sqf * (1.0 / two_n) - m * m
            S = g_ref[...] * lax.rsqrt(v + 1e-5)
            T = b_ref[...] - m * S
            r = jnp.maximum(h_ref[...] * S + T, 0.0)
        if mode == 2:
            out_ref = orefs[0]
            o = jnp.dot(r, Wo_ref[...], preferred_element_type=f32) + bo_ref[...]
            out_ref[...] = jax.nn.sigmoid(o)
        else:
            out_ref, stats_ref, acc_ref = orefs
            u = jnp.dot(r, W_ref[...], preferred_element_type=f32) + bh_ref[...]
            out_ref[...] = u
            @pl.when(i == 0)
            def _():
                acc_ref[...] = jnp.zeros_like(acc_ref)
            acc_ref[0:1, :] += jnp.sum(u, axis=0, keepdims=True)
            acc_ref[1:2, :] += jnp.sum(u * u, axis=0, keepdims=True)
            @pl.when(i == G - 1)
            def _():
                stats_ref[...] = acc_ref[...]

    full = lambda shape: pl.BlockSpec(shape, lambda i: tuple(0 for _ in shape))
    hspec = pl.BlockSpec((R, F), lambda i: (i, 0))
    if mode == 0:
        in_specs = [hspec, full((F, F)), full((1, F))]
        args = [hin, W16, bh16]
    elif mode == 1:
        in_specs = [hspec, full((F, F)), full((1, F)), full((F, F)),
                    full((8, F)), full((1, F)), full((1, F))]
        args = [hin, W16, bh16, P16, stats, g16, b16]
    else:
        in_specs = [hspec, full((F, F)), full((8, F)), full((1, F)),
                    full((1, F)), full((F, 8)), full((1, 8))]
        args = [hin, P16, stats, g16, b16, Wo16, bo2]
    if mode == 2:
        out_shape = jax.ShapeDtypeStruct((n, 8), f32)
        out_specs = pl.BlockSpec((R, 8), lambda i: (i, 0))
        scratch = []
    else:
        out_shape = [jax.ShapeDtypeStruct((n, F), f32),
                     jax.ShapeDtypeStruct((8, F), f32)]
        out_specs = [pl.BlockSpec((R, F), lambda i: (i, 0)),
                     pl.BlockSpec((8, F), lambda i: (0, 0))]
        scratch = [pltpu.VMEM((8, F), f32)]
    return pl.pallas_call(
        body,
        grid=(G,),
        in_specs=in_specs,
        out_specs=out_specs,
        out_shape=out_shape,
        scratch_shapes=scratch,
    )(*args)


def kernel(x, edge_index, Wl1, bl1, Wr1, Wl, bl, Wr, bn_g, bn_b,
           lin_W, lin_b, bn6_g, bn6_b, out_W, out_b):
    f32 = jnp.float32
    n = x.shape[0]
    e = edge_index.shape[1]
    n_pad = (n // 128 + 2) * 128
    ew = -(-e // (NW * CH)) * CH
    e_pad = ew * NW

    src = edge_index[0]
    dst = edge_index[1]
    src_flat = jnp.concatenate([src, jnp.zeros((e_pad - e,), jnp.int32)])
    dst_flat = jnp.concatenate([dst, jnp.full((e_pad - e,), n, jnp.int32)])
    zeros_stripe = jnp.zeros((n_pad // NS, F), f32)

    ecol = jnp.zeros((n, 1), f32)
    h16 = jnp.concatenate(
        [x, jnp.zeros((n, 12 - x.shape[1]), f32), 1.0 + ecol,
         jnp.zeros((n, 3), f32)], axis=1)

    def pad_w(w):
        return jnp.pad(w, ((0, F - w.shape[0]), (0, F - w.shape[1])))

    def pad_b(b, col12=0.0):
        v = jnp.pad(b, (0, F - b.shape[0]))
        return v.at[12].set(col12).reshape(1, F)

    def pad_v(v):
        return jnp.pad(v, (0, F - v.shape[0])).reshape(1, F)

    stats = None
    for i in range(8):
        agg2 = _aggregate(h16, src_flat, dst_flat, zeros_stripe, n_pad)
        if i == 0:
            Wl16, bl16, Wr16 = pad_w(Wl1), pad_b(bl1, 1.0), pad_w(Wr1)
            g16 = b16 = None
        else:
            Wl16 = pad_w(Wl[i - 1])
            bl16 = pad_b(bl[i - 1], 1.0)
            Wr16 = pad_w(Wr[i - 1])
            g16 = pad_v(bn_g[i - 1])
            b16 = pad_v(bn_b[i - 1])
        res = _conv_dense(agg2, h16, Wl16, bl16, Wr16, stats, g16, b16,
                          first=(i == 0), need_stats=(i < 7))
        if i < 7:
            h16, stats = res
        else:
            h16 = res

    Wh = np.zeros((F, F), np.float32)
    Wh16 = jnp.asarray(Wh).at[0:6, 0:6].set(lin_W).at[6:12, 6:12].set(lin_W)
    bh16 = jnp.concatenate([lin_b, lin_b, jnp.zeros((4,), f32)]).reshape(1, F)
    P = np.zeros((F, F), np.float32)
    for a in range(12):
        for bb in range(12):
            if a % 6 == bb % 6:
                P[a, bb] = 1.0
    P16 = jnp.asarray(P)
    g6_16 = jnp.concatenate([bn6_g, bn6_g, jnp.zeros((4,), f32)]).reshape(1, F)
    b6_16 = jnp.concatenate([bn6_b, bn6_b, jnp.zeros((4,), f32)]).reshape(1, F)
    Wo16 = jnp.zeros((F, 8), f32).at[0:6, 0:1].set(out_W).at[6:12, 1:2].set(out_W)
    bo2 = jnp.zeros((1, 8), f32).at[0, 0].set(out_b[0]).at[0, 1].set(out_b[0])

    u, hstats = _head(h16, Wh16, bh16, P16, None, None, None, None, None, mode=0)
    u, hstats = _head(u, Wh16, bh16, P16, hstats, g6_16, b6_16, None, None, mode=1)
    u, hstats = _head(u, Wh16, bh16, P16, hstats, g6_16, b6_16, None, None, mode=1)
    o = _head(u, None, None, P16, hstats, g6_16, b6_16, Wo16, bo2, mode=2)
    return o[:, 0:2].reshape(2 * n, 1)


# speedup vs baseline: 24.2359x; 1.4167x over previous
import functools

import jax
import jax.numpy as jnp
import numpy as np
from jax import lax
from jax.experimental import pallas as pl
from jax.experimental.pallas import tpu as pltpu
from jax.experimental.pallas import tpu_sc as plsc

F = 16
NC = 1
NS = 16
NW = NC * NS
EB = 128
KB = 56


CH = 512
U = 3


@functools.lru_cache(maxsize=None)
def _make_agg(n_pad, ew, stripe):
    mesh = plsc.VectorSubcoreMesh(core_axis_name="c", subcore_axis_name="s",
                                  num_cores=NC)
    nit = ew // CH
    nbody = nit // U
    ntail = nit - nbody * U

    @functools.partial(
        pl.kernel,
        out_type=jax.ShapeDtypeStruct((n_pad, F), jnp.float32),
        mesh=mesh,
        scratch_types=[
            pltpu.VMEM((U, CH), jnp.int32),
            pltpu.VMEM((U, CH), jnp.int32),
            pltpu.VMEM((U, CH, F), jnp.float32),
            pltpu.VMEM_SHARED((n_pad, F), jnp.float32),
        ] + [pltpu.SemaphoreType.DMA] * (3 * U),
        compiler_params=pltpu.CompilerParams(use_tc_tiling_on_sc=False),
    )
    def agg(h_hbm, src_hbm, dst_hbm, zero_hbm, out_hbm,
            src_v, dst_v, rows_v, acc_sh, *sems):
        semi = sems[0:U]
        semg = sems[U:2 * U]
        sems_ = sems[2 * U:3 * U]
        s = lax.axis_index("s")
        pltpu.sync_copy(zero_hbm, acc_sh.at[pl.ds(s * stripe, stripe)])
        plsc.subcore_barrier()
        base = s * ew

        def chunks(off, m):
            di = []
            for u in range(m):
                o = off + u * CH
                di.append((
                    pltpu.async_copy(src_hbm.at[pl.ds(o, CH)], src_v.at[u],
                                     semi[u]),
                    pltpu.async_copy(dst_hbm.at[pl.ds(o, CH)], dst_v.at[u],
                                     semi[u])))
            gd = []
            for u in range(m):
                di[u][0].wait()
                gd.append(pltpu.async_copy(h_hbm.at[src_v.at[u]],
                                           rows_v.at[u], semg[u]))
            sd = []
            for u in range(m):
                gd[u].wait()
                di[u][1].wait()
                sd.append(pltpu.async_copy(rows_v.at[u],
                                           acc_sh.at[dst_v.at[u]],
                                           sems_[u], add=True))
            for u in range(m):
                sd[u].wait()

        def body(ii, carry):
            chunks(base + ii * (U * CH), U)
            return carry

        lax.fori_loop(0, nbody, body, 0)
        if ntail:
            chunks(base + nbody * (U * CH), ntail)
        plsc.subcore_barrier()
        pltpu.sync_copy(acc_sh.at[pl.ds(s * stripe, stripe)],
                        out_hbm.at[pl.ds(s * stripe, stripe)])

    return agg


def _aggregate(h16, src_flat, dst_flat, zeros_stripe, n_pad):
    e_pad = src_flat.shape[0]
    agg = _make_agg(n_pad, e_pad // NW, n_pad // NS)
    return agg(h16, src_flat, dst_flat, zeros_stripe)


PK = 128
NPR = PK // F


def _conv_dense(agg_pk, h_pk, Wl128, bl128, Wr128, dspread, rept, rep,
                stats, g16, b16, n, first, need_stats):
    rows = h_pk.shape[0]
    f32 = jnp.float32
    vrows = n // NPR

    def body(*refs):
        if first:
            agg_ref, h_ref, Wl_ref, bl_ref, Wr_ref, ds_ref = refs[:6]
            orefs = refs[6:]
        else:
            (agg_ref, h_ref, Wl_ref, bl_ref, Wr_ref, ds_ref, rt_ref, rp_ref,
             st_ref, g_ref, b_ref) = refs[:11]
            orefs = refs[11:]
        if need_stats:
            out_ref, stats_ref = orefs
        else:
            (out_ref,) = orefs
        agg = agg_ref[...]
        deg = jnp.dot(agg, ds_ref[...], preferred_element_type=f32)
        mean = agg * (1.0 / jnp.maximum(deg, 1.0))
        if first:
            hn = h_ref[...]
            meanc = mean
        else:
            st = st_ref[...]
            s0 = jnp.dot(st[0:1, :], rt_ref[...], preferred_element_type=f32)
            s1 = jnp.dot(st[1:2, :], rt_ref[...], preferred_element_type=f32)
            m = s0 * (1.0 / n)
            v = s1 * (1.0 / n) - m * m
            sc16 = g_ref[...] * lax.rsqrt(v + 1e-5)
            tc16 = b_ref[...] - m * sc16
            sc = jnp.dot(sc16, rp_ref[...], preferred_element_type=f32)
            tc = jnp.dot(tc16, rp_ref[...], preferred_element_type=f32)
            hn = h_ref[...] * sc + tc
            meanc = mean * sc + tc * jnp.minimum(deg, 1.0)
        z = (jnp.dot(meanc, Wl_ref[...], preferred_element_type=f32)
             + bl_ref[...]
             + jnp.dot(hn, Wr_ref[...], preferred_element_type=f32))
        hp = jnp.maximum(z, 0.0)
        out_ref[...] = hp
        if need_stats:
            rid = lax.broadcasted_iota(jnp.int32, (rows, 1), 0)
            hm = jnp.where(rid < vrows, hp, 0.0)
            stats_ref[...] = jnp.concatenate(
                [jnp.sum(hm, axis=0, keepdims=True),
                 jnp.sum(hm * hm, axis=0, keepdims=True),
                 jnp.zeros((6, PK), f32)], axis=0)

    full = lambda shape: pl.BlockSpec(shape, lambda i: tuple(0 for _ in shape))
    blk = full((rows, PK))
    in_specs = [blk, blk, full((PK, PK)), full((1, PK)), full((PK, PK)),
                full((PK, PK))]
    args = [agg_pk, h_pk, Wl128, bl128, Wr128, dspread]
    if not first:
        in_specs += [full((PK, F)), full((F, PK)), full((8, PK)),
                     full((1, F)), full((1, F))]
        args += [rept, rep, stats, g16, b16]
    out_shape = [jax.ShapeDtypeStruct((rows, PK), f32)]
    out_specs = [blk]
    if need_stats:
        out_shape.append(jax.ShapeDtypeStruct((8, PK), f32))
        out_specs.append(full((8, PK)))
    res = pl.pallas_call(
        body,
        grid=(1,),
        in_specs=in_specs,
        out_specs=out_specs if len(out_specs) > 1 else out_specs[0],
        out_shape=out_shape if len(out_shape) > 1 else out_shape[0],
    )(*args)
    return res


def _head(hin, W128, bh128, P16, rept, rep, stats, g16, b16, Wo128, bo16,
          n, mode):
    rows = hin.shape[0]
    f32 = jnp.float32
    vrows = n // NPR
    two_n = 2.0 * n

    def body(*refs):
        if mode == 0:
            h_ref, W_ref, bh_ref = refs[:3]
            orefs = refs[3:]
        elif mode == 1:
            h_ref, W_ref, bh_ref, P_ref, rt_ref, rp_ref, st_ref, g_ref, b_ref = refs[:9]
            orefs = refs[9:]
        else:
            h_ref, P_ref, rt_ref, rp_ref, st_ref, g_ref, b_ref, Wo_ref, bo_ref = refs[:9]
            orefs = refs[9:]
        if mode == 0:
            r = h_ref[...]
        else:
            st = st_ref[...]
            s0 = jnp.dot(st[0:1, :], rt_ref[...], preferred_element_type=f32)
            s1 = jnp.dot(st[1:2, :], rt_ref[...], preferred_element_type=f32)
            sumf = jnp.dot(s0, P_ref[...], preferred_element_type=f32)
            sqf = jnp.dot(s1, P_ref[...], preferred_element_type=f32)
            m = sumf * (1.0 / two_n)
            v = sqf * (1.0 / two_n) - m * m
            S16 = g_ref[...] * lax.rsqrt(v + 1e-5)
            T16 = b_ref[...] - m * S16
            S = jnp.dot(S16, rp_ref[...], preferred_element_type=f32)
            T = jnp.dot(T16, rp_ref[...], preferred_element_type=f32)
            r = jnp.maximum(h_ref[...] * S + T, 0.0)
        if mode == 2:
            out_ref = orefs[0]
            o = jnp.dot(r, Wo_ref[...], preferred_element_type=f32) + bo_ref[...]
            out_ref[...] = jax.nn.sigmoid(o)
        else:
            out_ref, stats_ref = orefs
            u = jnp.dot(r, W_ref[...], preferred_element_type=f32) + bh_ref[...]
            out_ref[...] = u
            rid = lax.broadcasted_iota(jnp.int32, (rows, 1), 0)
            um = jnp.where(rid < vrows, u, 0.0)
            stats_ref[...] = jnp.concatenate(
                [jnp.sum(um, axis=0, keepdims=True),
                 jnp.sum(um * um, axis=0, keepdims=True),
                 jnp.zeros((6, PK), f32)], axis=0)

    full = lambda shape: pl.BlockSpec(shape, lambda i: tuple(0 for _ in shape))
    hspec = full((rows, PK))
    if mode == 0:
        in_specs = [hspec, full((PK, PK)), full((1, PK))]
        args = [hin, W128, bh128]
    elif mode == 1:
        in_specs = [hspec, full((PK, PK)), full((1, PK)), full((F, F)),
                    full((PK, F)), full((F, PK)), full((8, PK)),
                    full((1, F)), full((1, F))]
        args = [hin, W128, bh128, P16, rept, rep, stats, g16, b16]
    else:
        in_specs = [hspec, full((F, F)), full((PK, F)), full((F, PK)),
                    full((8, PK)), full((1, F)), full((1, F)),
                    full((PK, F)), full((1, F))]
        args = [hin, P16, rept, rep, stats, g16, b16, Wo128, bo16]
    if mode == 2:
        out_shape = jax.ShapeDtypeStruct((rows, F), f32)
        out_specs = full((rows, F))
    else:
        out_shape = [jax.ShapeDtypeStruct((rows, PK), f32),
                     jax.ShapeDtypeStruct((8, PK), f32)]
        out_specs = [hspec, full((8, PK))]
    return pl.pallas_call(
        body,
        grid=(1,),
        in_specs=in_specs,
        out_specs=out_specs,
        out_shape=out_shape,
    )(*args)


def kernel(x, edge_index, Wl1, bl1, Wr1, Wl, bl, Wr, bn_g, bn_b,
           lin_W, lin_b, bn6_g, bn6_b, out_W, out_b):
    f32 = jnp.float32
    n = x.shape[0]
    e = edge_index.shape[1]
    n_pad = (n // 128 + 2) * 128
    ew = -(-e // (NW * CH)) * CH
    e_pad = ew * NW

    src = edge_index[0]
    dst = edge_index[1]
    src_flat = jnp.concatenate([src, jnp.zeros((e_pad - e,), jnp.int32)])
    dst_flat = jnp.concatenate([dst, jnp.full((e_pad - e,), n, jnp.int32)])
    zeros_stripe = jnp.zeros((n_pad // NS, F), f32)

    ecol = jnp.zeros((n, 1), f32)
    h16 = jnp.concatenate(
        [x, jnp.zeros((n, 12 - x.shape[1]), f32), 1.0 + ecol,
         jnp.zeros((n, 3), f32)], axis=1)
    h16 = jnp.concatenate([h16, jnp.zeros((n_pad - n, F), f32)], axis=0)

    def pad_w(w):
        return jnp.pad(w, ((0, F - w.shape[0]), (0, F - w.shape[1])))

    def pad_b(b, col12=0.0):
        v = jnp.pad(b, (0, F - b.shape[0]))
        return v.at[12].set(col12).reshape(1, F)

    def pad_v(v):
        return jnp.pad(v, (0, F - v.shape[0])).reshape(1, F)

    eye8 = jnp.eye(NPR, dtype=f32)
    ones8 = jnp.ones((1, NPR), f32)

    def bd8(w16):
        return jnp.kron(eye8, w16)

    def tile8(v16):
        return jnp.kron(ones8, v16)

    rept = jnp.kron(jnp.ones((NPR, 1), f32), jnp.eye(F, dtype=f32))
    rep = jnp.kron(ones8, jnp.eye(F, dtype=f32))
    dsp = np.zeros((PK, PK), np.float32)
    for g in range(NPR):
        dsp[16 * g + 12, 16 * g:16 * g + 16] = 1.0
    dspread = jnp.asarray(dsp)

    stats = None
    for i in range(8):
        agg = _aggregate(h16, src_flat, dst_flat, zeros_stripe, n_pad)
        agg_pk = agg.reshape(n_pad // NPR, PK)
        h_pk = h16.reshape(n_pad // NPR, PK)
        if i == 0:
            Wl16, bl16, Wr16 = pad_w(Wl1), pad_b(bl1, 1.0), pad_w(Wr1)
            g16 = b16 = None
        else:
            Wl16 = pad_w(Wl[i - 1])
            bl16 = pad_b(bl[i - 1], 1.0)
            Wr16 = pad_w(Wr[i - 1])
            g16 = pad_v(bn_g[i - 1])
            b16 = pad_v(bn_b[i - 1])
        res = _conv_dense(agg_pk, h_pk, bd8(Wl16), tile8(bl16), bd8(Wr16),
                          dspread, rept, rep, stats, g16, b16, n,
                          first=(i == 0), need_stats=(i < 7))
        if i < 7:
            h_pk, stats = res
        else:
            h_pk = res
        h16 = h_pk.reshape(n_pad, F)

    Wh = np.zeros((F, F), np.float32)
    Wh16 = jnp.asarray(Wh).at[0:6, 0:6].set(lin_W).at[6:12, 6:12].set(lin_W)
    bh16 = jnp.concatenate([lin_b, lin_b, jnp.zeros((4,), f32)]).reshape(1, F)
    P = np.zeros((F, F), np.float32)
    for a in range(12):
        for bb in range(12):
            if a % 6 == bb % 6:
                P[a, bb] = 1.0
    P16 = jnp.asarray(P)
    g6_16 = jnp.concatenate([bn6_g, bn6_g, jnp.zeros((4,), f32)]).reshape(1, F)
    b6_16 = jnp.concatenate([bn6_b, bn6_b, jnp.zeros((4,), f32)]).reshape(1, F)
    Wo128 = jnp.zeros((PK, F), f32)
    for g in range(NPR):
        Wo128 = Wo128.at[16 * g:16 * g + 6, 2 * g:2 * g + 1].set(out_W)
        Wo128 = Wo128.at[16 * g + 6:16 * g + 12, 2 * g + 1:2 * g + 2].set(out_W)
    bo16 = jnp.full((1, F), out_b[0], f32)

    Wh128 = bd8(Wh16)
    bh128 = tile8(bh16)
    u, hstats = _head(h_pk, Wh128, bh128, P16, rept, rep, None, None, None,
                      None, None, n, mode=0)
    u, hstats = _head(u, Wh128, bh128, P16, rept, rep, hstats, g6_16, b6_16,
                      None, None, n, mode=1)
    u, hstats = _head(u, Wh128, bh128, P16, rept, rep, hstats, g6_16, b6_16,
                      None, None, n, mode=1)
    o = _head(u, None, None, P16, rept, rep, hstats, g6_16, b6_16,
              Wo128, bo16, n, mode=2)
    return o[:n // NPR].reshape(2 * n, 1)
